# baseline (speedup 1.0000x reference)
import math

import jax
import jax.numpy as jnp
import numpy as np
from jax.experimental import pallas as pl
from jax.experimental.pallas import tpu as pltpu

_EDGES = [float(np.float32(2.0 * math.pi) / np.float32(10.0) * np.float32(k))
          for k in range(1, 10)]
_COS = [float(np.float32(math.cos(e))) for e in _EDGES]
_SIN = [float(np.float32(math.sin(e))) for e in _EDGES]
_UPPER = [e > math.pi for e in _EDGES]

_D = 16
_FLAT_LANES = 2048
_BLOCK_ROWS = 512


def _wedge_counts_masked(v, ya, one_even, acc_ref):
    y_neg = ya < 0.0
    for k in range(9):
        cross = _COS[k] * ya - _SIN[k] * v
        c_pos = cross > 0.0
        if _UPPER[k]:
            pred = jnp.logical_and(y_neg, c_pos)
        else:
            pred = jnp.logical_or(y_neg, c_pos)
        acc_ref[k] += jnp.sum(jnp.where(pred, one_even, 0.0))


def _chi_square(nrows_total, o_ref, acc_ref):
    n_total = float(nrows_total)
    expected = float(nrows_total // 10)
    denom = expected + 1e-6
    t = [acc_ref[k] for k in range(9)]
    counts = [n_total - t[0]]
    counts += [t[j] - t[j + 1] for j in range(8)]
    counts += [t[8]]
    chi = (counts[0] - expected) ** 2 / denom
    for c in counts[1:]:
        chi = chi + (c - expected) ** 2 / denom
    o_ref[0] = chi


def kernel(x):
    n, d = x.shape
    flat_rows = (n * d) // _FLAT_LANES
    block_rows = min(_BLOCK_ROWS, flat_rows)
    grid = flat_rows // block_rows

    xf = x.reshape(flat_rows, _FLAT_LANES)

    def body(x_ref, o_ref, acc_ref):
        i = pl.program_id(0)

        @pl.when(i == 0)
        def _init():
            for k in range(9):
                acc_ref[k] = 0.0

        b = x_ref[...]
        lane = jax.lax.broadcasted_iota(jnp.int32, (block_rows, 128), 1)
        lane_mod = lane % _D

        chunks = [b[:, j * 128:(j + 1) * 128] for j in range(_D)]
        step = 2
        while len(chunks) > 2:
            keep = lane_mod < step
            chunks = [jnp.where(keep, a, pltpu.roll(c, step, 1))
                      for a, c in zip(chunks[0::2], chunks[1::2])]
            step *= 2

        even = (lane % 2) == 0
        c0, c1 = chunks
        xv = jnp.where(even, c0, pltpu.roll(c1, 127, 1))
        yv = jnp.where(even, pltpu.roll(c0, 127, 1), pltpu.roll(c1, 126, 1))
        _wedge_counts_masked(xv, yv, jnp.float32(1.0), acc_ref)

        @pl.when(i == grid - 1)
        def _fin():
            _chi_square(n, o_ref, acc_ref)

    out = pl.pallas_call(
        body,
        grid=(grid,),
        in_specs=[pl.BlockSpec((block_rows, _FLAT_LANES), lambda i: (i, 0))],
        out_specs=pl.BlockSpec(memory_space=pltpu.SMEM),
        out_shape=jax.ShapeDtypeStruct((1,), jnp.float32),
        scratch_shapes=[pltpu.SMEM((16,), jnp.float32)],
    )(xf)
    return out[0]
